# baseline (speedup 1.0000x reference)
import functools

import jax
import jax.numpy as jnp
from jax import lax
from jax.experimental import pallas as pl
from jax.experimental.pallas import tpu as pltpu
from jax.experimental.pallas import tpu_sc as plsc

NC = 2
NS = 16
NW = NC * NS
L = 16

N = 2_000_000
K = 1024

C0 = (N // NW) // L * L
TAIL = N - NW * C0
CT = C0 + TAIL

CH = 8192
_starts = list(range(0, CT, CH))
CHUNKS = [(o, min(CH, CT - o)) for o in _starts]
NCH = len(CHUNKS)
CHG = 16384
_gstarts = list(range(0, CT, CHG))
GCHUNKS = [(o, min(CHG, CT - o)) for o in _gstarts]
NCHG = len(GCHUNKS)
GWCHUNKS = [(o, min(CHG, C0 - o)) for o in _gstarts if o < C0]

UNROLL = 8

_mesh = plsc.VectorSubcoreMesh(core_axis_name="c", subcore_axis_name="s")
_sc_params = pltpu.CompilerParams(needs_layout_passes=False)


def _flags_call(labels):
    @functools.partial(
        pl.kernel,
        mesh=_mesh,
        out_type=jax.ShapeDtypeStruct((NW, K), jnp.int32),
        compiler_params=_sc_params,
        scratch_types=[
            pltpu.VMEM((CT,), jnp.int32),
            pltpu.VMEM((K,), jnp.int32),
            pltpu.VMEM((K,), jnp.int32),
            pltpu.VMEM((K,), jnp.int32),
            pltpu.VMEM((K,), jnp.int32),
        ]
        + [pltpu.SemaphoreType.DMA] * NCH,
    )
    def k(labels_hbm, flags_hbm, lab_v, flg_v, flg_w, flg_x, flg_y, *sems):
        c = lax.axis_index("c")
        s = lax.axis_index("s")
        wid = s * NC + c
        base = wid * C0

        def in_copy(j):
            off, sz = CHUNKS[j]
            return pltpu.make_async_copy(
                labels_hbm.at[pl.ds(base + off, sz)],
                lab_v.at[pl.ds(off, sz)],
                sems[j],
            )

        for j in range(NCH):
            in_copy(j).start()

        zeros = jnp.zeros((L,), jnp.int32)
        for i in range(K // L):
            flg_v[pl.ds(i * L, L)] = zeros
            flg_w[pl.ds(i * L, L)] = zeros
            flg_x[pl.ds(i * L, L)] = zeros
            flg_y[pl.ds(i * L, L)] = zeros

        ones = jnp.ones((L,), jnp.int32)
        for j in range(NCH):
            off, sz = CHUNKS[j]
            in_copy(j).wait()

            def quad(i, carry, off=off):
                lv0 = lab_v[pl.ds(off + i * 4 * L, L)]
                lv1 = lab_v[pl.ds(off + i * 4 * L + L, L)]
                lv2 = lab_v[pl.ds(off + i * 4 * L + 2 * L, L)]
                lv3 = lab_v[pl.ds(off + i * 4 * L + 3 * L, L)]
                plsc.store_scatter(flg_v, [lv0 - 1], ones)
                plsc.store_scatter(flg_w, [lv1 - 1], ones)
                plsc.store_scatter(flg_x, [lv2 - 1], ones)
                plsc.store_scatter(flg_y, [lv3 - 1], ones)
                return carry

            nq = sz // (4 * L)
            lax.fori_loop(0, nq, quad, 0, unroll=2)
            for r in range(nq * 4 * L, sz, 2 * L):
                lv0 = lab_v[pl.ds(off + r, L)]
                lv1 = lab_v[pl.ds(off + r + L, L)]
                plsc.store_scatter(flg_v, [lv0 - 1], ones)
                plsc.store_scatter(flg_w, [lv1 - 1], ones)

        for i in range(K // L):
            flg_v[pl.ds(i * L, L)] = (
                flg_v[pl.ds(i * L, L)] + flg_w[pl.ds(i * L, L)]
            ) + (flg_x[pl.ds(i * L, L)] + flg_y[pl.ds(i * L, L)])

        pltpu.sync_copy(flg_v, flags_hbm.at[wid])

    return k(labels)


def _rank_body(flags_ref, peak_row_ref, peak_col_ref, out_ref):
    occ_row = (jnp.sum(flags_ref[...], axis=0, keepdims=True) > 0).astype(
        jnp.float32
    )
    kidx_row = lax.broadcasted_iota(jnp.int32, (1, K), 1)
    kmax = jnp.max(jnp.where(occ_row > 0, kidx_row + 1, 0))

    ii = lax.broadcasted_iota(jnp.int32, (K, K), 0)
    jj = lax.broadcasted_iota(jnp.int32, (K, K), 1)
    iden = (ii == jj).astype(jnp.float32)
    occ_col = lax.dot_general(
        iden,
        occ_row,
        (((1,), (1,)), ((), ())),
        preferred_element_type=jnp.float32,
    )

    inf = jnp.float32(jnp.inf)
    aa_row = jnp.where(
        jnp.logical_and(occ_row == 0.0, kidx_row < kmax), inf, peak_row_ref[...]
    )
    kidx_col = lax.broadcasted_iota(jnp.int32, (K, 1), 0)
    aa_col = jnp.where(
        jnp.logical_and(occ_col == 0.0, kidx_col < kmax), inf, peak_col_ref[...]
    )

    before = jnp.logical_or(
        aa_row < aa_col, jnp.logical_and(aa_row == aa_col, jj < ii)
    )
    rank = jnp.sum(before.astype(jnp.int32), axis=1, keepdims=True)
    out_ref[...] = jnp.broadcast_to(rank + 1, (K, L))


def _rank_call(flags, peak):
    return pl.pallas_call(
        _rank_body,
        out_shape=jax.ShapeDtypeStruct((K, L), jnp.int32),
    )(flags, peak.reshape(1, K), peak.reshape(K, 1))


def _gather_call(labels, mapping):
    @functools.partial(
        pl.kernel,
        mesh=_mesh,
        out_type=jax.ShapeDtypeStruct((N,), jnp.int32),
        compiler_params=_sc_params,
        scratch_types=[
            pltpu.VMEM((CT,), jnp.int32),
            pltpu.VMEM((K * L,), jnp.int32),
        ]
        + [pltpu.SemaphoreType.DMA] * (2 * NCHG + 1),
    )
    def k(labels_hbm, map_hbm, out_hbm, lab_v, tabr_v, *sems):
        c = lax.axis_index("c")
        s = lax.axis_index("s")
        wid = s * NC + c
        base = wid * C0
        sems_in = sems[:NCHG]
        sems_out = sems[NCHG : 2 * NCHG]
        sem_tail = sems[2 * NCHG]

        def in_copy(j):
            off, sz = GCHUNKS[j]
            return pltpu.make_async_copy(
                labels_hbm.at[pl.ds(base + off, sz)],
                lab_v.at[pl.ds(off, sz)],
                sems_in[j],
            )

        def out_copy(j):
            off, sz = GWCHUNKS[j]
            return pltpu.make_async_copy(
                lab_v.at[pl.ds(off, sz)],
                out_hbm.at[pl.ds(base + off, sz)],
                sems_out[j],
            )

        def tail_copy():
            return pltpu.make_async_copy(
                lab_v.at[pl.ds(C0, TAIL)],
                out_hbm.at[pl.ds(base + C0, TAIL)],
                sem_tail,
            )

        tab_cp = pltpu.make_async_copy(map_hbm, tabr_v, sem_tail)
        tab_cp.start()
        for j in range(NCHG):
            in_copy(j).start()
        tab_cp.wait()

        lane_off = lax.broadcasted_iota(jnp.int32, (L,), 0) - L

        for j in range(NCHG):
            off, sz = GCHUNKS[j]
            in_copy(j).wait()

            def body(i, carry, off=off):
                lv = lab_v[pl.ds(off + i * L, L)]
                idx = jnp.left_shift(lv, 4) + lane_off
                lab_v[pl.ds(off + i * L, L)] = plsc.load_gather(
                    tabr_v, [idx]
                )
                return carry

            lax.fori_loop(0, sz // L, body, 0, unroll=UNROLL)
            out_copy(j).start()

        @pl.when(wid == NW - 1)
        def _():
            tail_copy().start()
            tail_copy().wait()

        for j in range(NCHG):
            out_copy(j).wait()

    return k(labels, mapping)


def kernel(labels, peak_channel_indices):
    flags = _flags_call(labels)
    mapping = _rank_call(flags, peak_channel_indices)
    return _gather_call(labels, mapping.reshape(K * L))


# speedup vs baseline: 1.0097x; 1.0097x over previous
import functools

import jax
import jax.numpy as jnp
from jax import lax
from jax.experimental import pallas as pl
from jax.experimental.pallas import tpu as pltpu
from jax.experimental.pallas import tpu_sc as plsc

NC = 2
NS = 16
NW = NC * NS
L = 16

N = 2_000_000
K = 1024

C0 = (N // NW) // L * L
TAIL = N - NW * C0
CT = C0 + TAIL

CH = 8192
_starts = list(range(0, CT, CH))
CHUNKS = [(o, min(CH, CT - o)) for o in _starts]
NCH = len(CHUNKS)
CHG = 16384
CHG0 = 4096
_gstarts = [0] + list(range(CHG0, CT, CHG))
GCHUNKS = [
    (o, min((CHG0 if o == 0 else CHG), CT - o)) for o in _gstarts
]
NCHG = len(GCHUNKS)
GWCHUNKS = [
    (o, min((CHG0 if o == 0 else CHG), C0 - o)) for o in _gstarts if o < C0
]

UNROLL = 8

_mesh = plsc.VectorSubcoreMesh(core_axis_name="c", subcore_axis_name="s")
_sc_params = pltpu.CompilerParams(needs_layout_passes=False)


def _flags_call(labels):
    @functools.partial(
        pl.kernel,
        mesh=_mesh,
        out_type=jax.ShapeDtypeStruct((NW, K), jnp.int32),
        compiler_params=_sc_params,
        scratch_types=[
            pltpu.VMEM((CT,), jnp.int32),
            pltpu.VMEM((K,), jnp.int32),
            pltpu.VMEM((K,), jnp.int32),
            pltpu.VMEM((K,), jnp.int32),
            pltpu.VMEM((K,), jnp.int32),
        ]
        + [pltpu.SemaphoreType.DMA] * NCH,
    )
    def k(labels_hbm, flags_hbm, lab_v, flg_v, flg_w, flg_x, flg_y, *sems):
        c = lax.axis_index("c")
        s = lax.axis_index("s")
        wid = s * NC + c
        base = wid * C0

        def in_copy(j):
            off, sz = CHUNKS[j]
            return pltpu.make_async_copy(
                labels_hbm.at[pl.ds(base + off, sz)],
                lab_v.at[pl.ds(off, sz)],
                sems[j],
            )

        for j in range(NCH):
            in_copy(j).start()

        zeros = jnp.zeros((L,), jnp.int32)
        for i in range(K // L):
            flg_v[pl.ds(i * L, L)] = zeros
            flg_w[pl.ds(i * L, L)] = zeros
            flg_x[pl.ds(i * L, L)] = zeros
            flg_y[pl.ds(i * L, L)] = zeros

        ones = jnp.ones((L,), jnp.int32)
        for j in range(NCH):
            off, sz = CHUNKS[j]
            in_copy(j).wait()

            def quad(i, carry, off=off):
                lv0 = lab_v[pl.ds(off + i * 4 * L, L)]
                lv1 = lab_v[pl.ds(off + i * 4 * L + L, L)]
                lv2 = lab_v[pl.ds(off + i * 4 * L + 2 * L, L)]
                lv3 = lab_v[pl.ds(off + i * 4 * L + 3 * L, L)]
                plsc.store_scatter(flg_v, [lv0 - 1], ones)
                plsc.store_scatter(flg_w, [lv1 - 1], ones)
                plsc.store_scatter(flg_x, [lv2 - 1], ones)
                plsc.store_scatter(flg_y, [lv3 - 1], ones)
                return carry

            nq = sz // (4 * L)
            lax.fori_loop(0, nq, quad, 0, unroll=2)
            for r in range(nq * 4 * L, sz, 2 * L):
                lv0 = lab_v[pl.ds(off + r, L)]
                lv1 = lab_v[pl.ds(off + r + L, L)]
                plsc.store_scatter(flg_v, [lv0 - 1], ones)
                plsc.store_scatter(flg_w, [lv1 - 1], ones)

        for i in range(K // L):
            flg_v[pl.ds(i * L, L)] = (
                flg_v[pl.ds(i * L, L)] + flg_w[pl.ds(i * L, L)]
            ) + (flg_x[pl.ds(i * L, L)] + flg_y[pl.ds(i * L, L)])

        pltpu.sync_copy(flg_v, flags_hbm.at[wid])

    return k(labels)


def _rank_body(flags_ref, peak_row_ref, peak_col_ref, out_ref):
    occ_row = (jnp.sum(flags_ref[...], axis=0, keepdims=True) > 0).astype(
        jnp.float32
    )
    kidx_row = lax.broadcasted_iota(jnp.int32, (1, K), 1)
    kmax = jnp.max(jnp.where(occ_row > 0, kidx_row + 1, 0))

    ii = lax.broadcasted_iota(jnp.int32, (K, K), 0)
    jj = lax.broadcasted_iota(jnp.int32, (K, K), 1)
    iden = (ii == jj).astype(jnp.float32)
    occ_col = lax.dot_general(
        iden,
        occ_row,
        (((1,), (1,)), ((), ())),
        preferred_element_type=jnp.float32,
    )

    inf = jnp.float32(jnp.inf)
    aa_row = jnp.where(
        jnp.logical_and(occ_row == 0.0, kidx_row < kmax), inf, peak_row_ref[...]
    )
    kidx_col = lax.broadcasted_iota(jnp.int32, (K, 1), 0)
    aa_col = jnp.where(
        jnp.logical_and(occ_col == 0.0, kidx_col < kmax), inf, peak_col_ref[...]
    )

    before = jnp.logical_or(
        aa_row < aa_col, jnp.logical_and(aa_row == aa_col, jj < ii)
    )
    rank = jnp.sum(before.astype(jnp.int32), axis=1, keepdims=True)
    out_ref[...] = jnp.broadcast_to(rank + 1, (K, L))


def _rank_call(flags, peak):
    return pl.pallas_call(
        _rank_body,
        out_shape=jax.ShapeDtypeStruct((K, L), jnp.int32),
    )(flags, peak.reshape(1, K), peak.reshape(K, 1))


def _gather_call(labels, mapping):
    @functools.partial(
        pl.kernel,
        mesh=_mesh,
        out_type=jax.ShapeDtypeStruct((N,), jnp.int32),
        compiler_params=_sc_params,
        scratch_types=[
            pltpu.VMEM((CT,), jnp.int32),
            pltpu.VMEM((K * L,), jnp.int32),
        ]
        + [pltpu.SemaphoreType.DMA] * (2 * NCHG + 1),
    )
    def k(labels_hbm, map_hbm, out_hbm, lab_v, tabr_v, *sems):
        c = lax.axis_index("c")
        s = lax.axis_index("s")
        wid = s * NC + c
        base = wid * C0
        sems_in = sems[:NCHG]
        sems_out = sems[NCHG : 2 * NCHG]
        sem_tail = sems[2 * NCHG]

        def in_copy(j):
            off, sz = GCHUNKS[j]
            return pltpu.make_async_copy(
                labels_hbm.at[pl.ds(base + off, sz)],
                lab_v.at[pl.ds(off, sz)],
                sems_in[j],
            )

        def out_copy(j):
            off, sz = GWCHUNKS[j]
            return pltpu.make_async_copy(
                lab_v.at[pl.ds(off, sz)],
                out_hbm.at[pl.ds(base + off, sz)],
                sems_out[j],
            )

        def tail_copy():
            return pltpu.make_async_copy(
                lab_v.at[pl.ds(C0, TAIL)],
                out_hbm.at[pl.ds(base + C0, TAIL)],
                sem_tail,
            )

        for j in range(NCHG):
            in_copy(j).start()
        pltpu.sync_copy(map_hbm, tabr_v)

        lane_off = lax.broadcasted_iota(jnp.int32, (L,), 0) - L

        for j in range(NCHG):
            off, sz = GCHUNKS[j]
            in_copy(j).wait()

            def body(i, carry, off=off):
                lv = lab_v[pl.ds(off + i * L, L)]
                idx = jnp.left_shift(lv, 4) + lane_off
                lab_v[pl.ds(off + i * L, L)] = plsc.load_gather(
                    tabr_v, [idx]
                )
                return carry

            lax.fori_loop(0, sz // L, body, 0, unroll=2 * UNROLL)
            out_copy(j).start()

        @pl.when(wid == NW - 1)
        def _():
            tail_copy().start()
            tail_copy().wait()

        for j in range(NCHG):
            out_copy(j).wait()

    return k(labels, mapping)


def kernel(labels, peak_channel_indices):
    flags = _flags_call(labels)
    mapping = _rank_call(flags, peak_channel_indices)
    return _gather_call(labels, mapping.reshape(K * L))


# speedup vs baseline: 1.0185x; 1.0087x over previous
import functools

import jax
import jax.numpy as jnp
from jax import lax
from jax.experimental import pallas as pl
from jax.experimental.pallas import tpu as pltpu
from jax.experimental.pallas import tpu_sc as plsc

NC = 2
NS = 16
NW = NC * NS
L = 16

N = 2_000_000
K = 1024

C0 = (N // NW) // L * L
TAIL = N - NW * C0
CT = C0 + TAIL

CH = 8192
_starts = list(range(0, CT, CH))
CHUNKS = [(o, min(CH, CT - o)) for o in _starts]
NCH = len(CHUNKS)
CHG = 16384
_gstarts = list(range(0, CT, CHG))
GCHUNKS = [(o, min(CHG, CT - o)) for o in _gstarts]
NCHG = len(GCHUNKS)
GWCHUNKS = [(o, min(CHG, C0 - o)) for o in _gstarts if o < C0]

UNROLL = 8

_mesh = plsc.VectorSubcoreMesh(core_axis_name="c", subcore_axis_name="s")
_sc_params = pltpu.CompilerParams(needs_layout_passes=False)


def _flags_call(labels):
    @functools.partial(
        pl.kernel,
        mesh=_mesh,
        out_type=jax.ShapeDtypeStruct((NW, K), jnp.int32),
        compiler_params=_sc_params,
        scratch_types=[
            pltpu.VMEM((CT,), jnp.int32),
            pltpu.VMEM((K,), jnp.int32),
            pltpu.VMEM((K,), jnp.int32),
            pltpu.VMEM((K,), jnp.int32),
            pltpu.VMEM((K,), jnp.int32),
        ]
        + [pltpu.SemaphoreType.DMA] * NCH,
    )
    def k(labels_hbm, flags_hbm, lab_v, flg_v, flg_w, flg_x, flg_y, *sems):
        c = lax.axis_index("c")
        s = lax.axis_index("s")
        wid = s * NC + c
        base = wid * C0

        def in_copy(j):
            off, sz = CHUNKS[j]
            return pltpu.make_async_copy(
                labels_hbm.at[pl.ds(base + off, sz)],
                lab_v.at[pl.ds(off, sz)],
                sems[j],
            )

        for j in range(NCH):
            in_copy(j).start()

        zeros = jnp.zeros((L,), jnp.int32)
        for i in range(K // L):
            flg_v[pl.ds(i * L, L)] = zeros
            flg_w[pl.ds(i * L, L)] = zeros
            flg_x[pl.ds(i * L, L)] = zeros
            flg_y[pl.ds(i * L, L)] = zeros

        ones = jnp.ones((L,), jnp.int32)
        for j in range(NCH):
            off, sz = CHUNKS[j]
            in_copy(j).wait()

            def quad(i, carry, off=off):
                lv0 = lab_v[pl.ds(off + i * 4 * L, L)]
                lv1 = lab_v[pl.ds(off + i * 4 * L + L, L)]
                lv2 = lab_v[pl.ds(off + i * 4 * L + 2 * L, L)]
                lv3 = lab_v[pl.ds(off + i * 4 * L + 3 * L, L)]
                plsc.store_scatter(flg_v, [lv0 - 1], ones)
                plsc.store_scatter(flg_w, [lv1 - 1], ones)
                plsc.store_scatter(flg_x, [lv2 - 1], ones)
                plsc.store_scatter(flg_y, [lv3 - 1], ones)
                return carry

            nq = sz // (4 * L)
            lax.fori_loop(0, nq, quad, 0, unroll=2)
            for r in range(nq * 4 * L, sz, 2 * L):
                lv0 = lab_v[pl.ds(off + r, L)]
                lv1 = lab_v[pl.ds(off + r + L, L)]
                plsc.store_scatter(flg_v, [lv0 - 1], ones)
                plsc.store_scatter(flg_w, [lv1 - 1], ones)

        for i in range(K // L):
            flg_v[pl.ds(i * L, L)] = (
                flg_v[pl.ds(i * L, L)] + flg_w[pl.ds(i * L, L)]
            ) + (flg_x[pl.ds(i * L, L)] + flg_y[pl.ds(i * L, L)])

        pltpu.sync_copy(flg_v, flags_hbm.at[wid])

    return k(labels)


def _rank_body(flags_ref, peak_row_ref, peak_col_ref, out_ref):
    occ_row = (jnp.sum(flags_ref[...], axis=0, keepdims=True) > 0).astype(
        jnp.float32
    )
    kidx_row = lax.broadcasted_iota(jnp.int32, (1, K), 1)
    kmax = jnp.max(jnp.where(occ_row > 0, kidx_row + 1, 0))

    ii = lax.broadcasted_iota(jnp.int32, (K, K), 0)
    jj = lax.broadcasted_iota(jnp.int32, (K, K), 1)
    iden = (ii == jj).astype(jnp.float32)
    occ_col = lax.dot_general(
        iden,
        occ_row,
        (((1,), (1,)), ((), ())),
        preferred_element_type=jnp.float32,
    )

    inf = jnp.float32(jnp.inf)
    aa_row = jnp.where(
        jnp.logical_and(occ_row == 0.0, kidx_row < kmax), inf, peak_row_ref[...]
    )
    kidx_col = lax.broadcasted_iota(jnp.int32, (K, 1), 0)
    aa_col = jnp.where(
        jnp.logical_and(occ_col == 0.0, kidx_col < kmax), inf, peak_col_ref[...]
    )

    before = jnp.logical_or(
        aa_row < aa_col, jnp.logical_and(aa_row == aa_col, jj < ii)
    )
    rank = jnp.sum(before.astype(jnp.int32), axis=1, keepdims=True)
    out_ref[...] = jnp.broadcast_to(rank + 1, (K, L))


def _rank_call(flags, peak):
    return pl.pallas_call(
        _rank_body,
        out_shape=jax.ShapeDtypeStruct((K, L), jnp.int32),
    )(flags, peak.reshape(1, K), peak.reshape(K, 1))


def _gather_call(labels, mapping):
    @functools.partial(
        pl.kernel,
        mesh=_mesh,
        out_type=jax.ShapeDtypeStruct((N,), jnp.int32),
        compiler_params=_sc_params,
        scratch_types=[
            pltpu.VMEM((CT,), jnp.int32),
            pltpu.VMEM((K * L,), jnp.int32),
        ]
        + [pltpu.SemaphoreType.DMA] * (2 * NCHG + 1),
    )
    def k(labels_hbm, map_hbm, out_hbm, lab_v, tabr_v, *sems):
        c = lax.axis_index("c")
        s = lax.axis_index("s")
        wid = s * NC + c
        base = wid * C0
        sems_in = sems[:NCHG]
        sems_out = sems[NCHG : 2 * NCHG]
        sem_tail = sems[2 * NCHG]

        def in_copy(j):
            off, sz = GCHUNKS[j]
            return pltpu.make_async_copy(
                labels_hbm.at[pl.ds(base + off, sz)],
                lab_v.at[pl.ds(off, sz)],
                sems_in[j],
            )

        def out_copy(j):
            off, sz = GWCHUNKS[j]
            return pltpu.make_async_copy(
                lab_v.at[pl.ds(off, sz)],
                out_hbm.at[pl.ds(base + off, sz)],
                sems_out[j],
            )

        def tail_copy():
            return pltpu.make_async_copy(
                lab_v.at[pl.ds(C0, TAIL)],
                out_hbm.at[pl.ds(base + C0, TAIL)],
                sem_tail,
            )

        for j in range(NCHG):
            in_copy(j).start()
        pltpu.sync_copy(map_hbm, tabr_v)

        lane_off = lax.broadcasted_iota(jnp.int32, (L,), 0) - L

        for j in range(NCHG):
            off, sz = GCHUNKS[j]
            in_copy(j).wait()

            def body(i, carry, off=off):
                lv = lab_v[pl.ds(off + i * L, L)]
                idx = jnp.left_shift(lv, 4) + lane_off
                lab_v[pl.ds(off + i * L, L)] = plsc.load_gather(
                    tabr_v, [idx]
                )
                return carry

            lax.fori_loop(0, sz // L, body, 0, unroll=UNROLL)
            out_copy(j).start()

        @pl.when(wid == NW - 1)
        def _():
            tail_copy().start()
            tail_copy().wait()

        for j in range(NCHG):
            out_copy(j).wait()

    return k(labels, mapping)


def kernel(labels, peak_channel_indices):
    flags = _flags_call(labels)
    mapping = _rank_call(flags, peak_channel_indices)
    return _gather_call(labels, mapping.reshape(K * L))
